# baseline (speedup 1.0000x reference)
import functools

import jax
import jax.numpy as jnp
from jax import lax
from jax.experimental import pallas as pl
from jax.experimental.pallas import tpu as pltpu
from jax.experimental.pallas import tpu_sc as plsc

BATCH = 4096
SEQ = 200
DIM = 64
LANES = 16
NW = 32
BBLK = BATCH // NW
PITCH = 133
NGBUF = 5


def _sc_body(idx_hbm, table_hbm, pos_hbm, out_hbm,
             idx_v, rows_v, brick_v, pos_v, *sems):
    cid = lax.axis_index("c")
    sid = lax.axis_index("s")
    wid = sid * 2 + cid
    sem_g = sems[:NGBUF]
    sem_o = sems[NGBUF:]

    pltpu.sync_copy(pos_hbm, pos_v)
    pltpu.sync_copy(idx_hbm.at[:, pl.ds(wid, 1)], idx_v)

    def fire_gather(s, r):
        pltpu.async_copy(
            table_hbm.at[idx_v.at[s // 8, 0, s % 8]],
            rows_v.at[r],
            sem_g[r],
        )

    def drain_gather(r):
        pltpu.make_async_copy(
            table_hbm.at[pl.ds(0, BBLK)], rows_v.at[r], sem_g[r]
        ).wait()

    def drain_store(p):
        pltpu.make_async_copy(
            brick_v.at[p].at[:, :, pl.ds(0, 128)],
            out_hbm.at[0, :, 0],
            sem_o[p],
        ).wait()

    lane_d = [jnp.arange(k * LANES, (k + 1) * LANES, dtype=jnp.int32)
              for k in range(DIM // LANES)]
    dt_vecs = [d // 8 for d in lane_d]
    di_vecs = [d % 8 for d in lane_d]

    def build_brick(s, r, p):
        rows = rows_v.at[r]
        brick = brick_v.at[p]
        pvs = [pos_v[s, pl.ds(k * LANES, LANES)] for k in range(DIM // LANES)]

        @plsc.parallel_loop(0, BBLK, unroll=8)
        def _(b):
            col = jnp.full((LANES,), b, dtype=jnp.int32)
            for k in range(DIM // LANES):
                v = rows[b, pl.ds(k * LANES, LANES)] + pvs[k]
                plsc.store_scatter(brick, [dt_vecs[k], di_vecs[k], col], v)

    for s0 in range(NGBUF - 1):
        fire_gather(s0, s0)

    @pl.loop(0, SEQ, step=NGBUF)
    def _(ss):
        for r in range(NGBUF):
            s = ss + r

            @pl.when(s + NGBUF - 1 < SEQ)
            def _prefetch():
                fire_gather(s + NGBUF - 1, (r + NGBUF - 1) % NGBUF)

            drain_gather(r)

            @pl.when(s >= NGBUF)
            def _reclaim():
                drain_store(r)

            build_brick(s, r, r)
            pltpu.async_copy(
                brick_v.at[r].at[:, :, pl.ds(0, 128)],
                out_hbm.at[s, :, wid],
                sem_o[r],
            )

    for r in range(NGBUF):
        drain_store(r)


_sc_embed = functools.partial(
    pl.kernel,
    out_type=jax.ShapeDtypeStruct((SEQ, DIM // 8, NW, 8, 128), jnp.float32),
    mesh=plsc.VectorSubcoreMesh(
        core_axis_name="c", subcore_axis_name="s", num_cores=2, num_subcores=16
    ),
    scratch_types=[
        pltpu.VMEM((SEQ // 8, 1, 8, 128), jnp.int32),
        pltpu.VMEM((NGBUF, BBLK, DIM), jnp.float32),
        pltpu.VMEM((NGBUF, DIM // 8, 8, PITCH), jnp.float32),
        pltpu.VMEM((SEQ, DIM), jnp.float32),
    ] + [pltpu.SemaphoreType.DMA] * (2 * NGBUF),
    compiler_params=pltpu.CompilerParams(
        use_tc_tiling_on_sc=False, needs_layout_passes=False
    ),
)(_sc_body)


def kernel(inputs, table, pos_table):
    idx = (inputs.transpose(1, 0).reshape(SEQ // 8, 8, NW, 128)
           .transpose(0, 2, 1, 3))
    q = _sc_embed(idx, table, pos_table)
    return q.transpose(2, 4, 0, 1, 3).reshape(BATCH, SEQ, DIM)


# speedup vs baseline: 1.0252x; 1.0252x over previous
import functools

import jax
import jax.numpy as jnp
from jax import lax
from jax.experimental import pallas as pl
from jax.experimental.pallas import tpu as pltpu
from jax.experimental.pallas import tpu_sc as plsc

BATCH = 4096
SEQ = 200
DIM = 64
LANES = 16
NW = 32
BBLK = BATCH // NW
PITCH = 133
NGBUF = 4


def _sc_body(idx_hbm, table_hbm, pos_hbm, out_hbm,
             idx_v, rows_v, brick_v, pos_v,
             sem_g0, sem_g1, sem_g2, sem_g3, sem_o0, sem_o1):
    cid = lax.axis_index("c")
    sid = lax.axis_index("s")
    wid = sid * 2 + cid
    sem_g = (sem_g0, sem_g1, sem_g2, sem_g3)
    sem_o = (sem_o0, sem_o1)

    pltpu.sync_copy(pos_hbm, pos_v)
    pltpu.sync_copy(idx_hbm.at[:, pl.ds(wid, 1)], idx_v)

    def fire_gather(s, r):
        pltpu.async_copy(
            table_hbm.at[idx_v.at[s // 8, 0, s % 8]],
            rows_v.at[r],
            sem_g[r],
        )

    def drain_gather(r):
        pltpu.make_async_copy(
            table_hbm.at[pl.ds(0, BBLK)], rows_v.at[r], sem_g[r]
        ).wait()

    def drain_store(p):
        pltpu.make_async_copy(
            brick_v.at[p].at[:, :, pl.ds(0, 128)],
            out_hbm.at[0, :, 0],
            sem_o[p],
        ).wait()

    lane_d = [jnp.arange(k * LANES, (k + 1) * LANES, dtype=jnp.int32)
              for k in range(DIM // LANES)]
    dt_vecs = [d // 8 for d in lane_d]
    di_vecs = [d % 8 for d in lane_d]

    def build_brick(s, r, p):
        rows = rows_v.at[r]
        brick = brick_v.at[p]
        pvs = [pos_v[s, pl.ds(k * LANES, LANES)] for k in range(DIM // LANES)]

        @plsc.parallel_loop(0, BBLK, unroll=8)
        def _(b):
            col = jnp.full((LANES,), b, dtype=jnp.int32)
            for k in range(DIM // LANES):
                v = rows[b, pl.ds(k * LANES, LANES)] + pvs[k]
                plsc.store_scatter(brick, [dt_vecs[k], di_vecs[k], col], v)

    for s0 in range(3):
        fire_gather(s0, s0)

    @pl.loop(0, SEQ, step=NGBUF)
    def _(ss):
        for r in range(NGBUF):
            s = ss + r
            p = r % 2

            @pl.when(s + 3 < SEQ)
            def _prefetch():
                fire_gather(s + 3, (r + 3) % NGBUF)

            drain_gather(r)

            @pl.when(s >= 2)
            def _reclaim():
                drain_store(p)

            build_brick(s, r, p)
            pltpu.async_copy(
                brick_v.at[p].at[:, :, pl.ds(0, 128)],
                out_hbm.at[s, :, wid],
                sem_o[p],
            )

    drain_store(0)
    drain_store(1)


_sc_embed = functools.partial(
    pl.kernel,
    out_type=jax.ShapeDtypeStruct((SEQ, DIM // 8, NW, 8, 128), jnp.float32),
    mesh=plsc.VectorSubcoreMesh(
        core_axis_name="c", subcore_axis_name="s", num_cores=2, num_subcores=16
    ),
    scratch_types=[
        pltpu.VMEM((SEQ // 8, 1, 8, 128), jnp.int32),
        pltpu.VMEM((NGBUF, BBLK, DIM), jnp.float32),
        pltpu.VMEM((2, DIM // 8, 8, PITCH), jnp.float32),
        pltpu.VMEM((SEQ, DIM), jnp.float32),
        pltpu.SemaphoreType.DMA,
        pltpu.SemaphoreType.DMA,
        pltpu.SemaphoreType.DMA,
        pltpu.SemaphoreType.DMA,
        pltpu.SemaphoreType.DMA,
        pltpu.SemaphoreType.DMA,
    ],
    compiler_params=pltpu.CompilerParams(
        use_tc_tiling_on_sc=False, needs_layout_passes=False
    ),
)(_sc_body)


def kernel(inputs, table, pos_table):
    idx = (inputs.transpose(1, 0).reshape(SEQ // 8, 8, NW, 128)
           .transpose(0, 2, 1, 3))
    q = _sc_embed(idx, table, pos_table)
    return q.transpose(2, 4, 0, 1, 3).reshape(BATCH, SEQ, DIM)


# speedup vs baseline: 1.0873x; 1.0606x over previous
import functools

import jax
import jax.numpy as jnp
from jax import lax
from jax.experimental import pallas as pl
from jax.experimental.pallas import tpu as pltpu
from jax.experimental.pallas import tpu_sc as plsc

BATCH = 4096
SEQ = 200
DIM = 64
LANES = 16
NW = 32
BBLK = BATCH // NW
PITCH = 133
NGBUF = 4


def _sc_body(idx_hbm, table_hbm, pos_hbm, out_hbm,
             idx_v, rows_v, brick_v, pos_v,
             sem_g0, sem_g1, sem_g2, sem_g3, sem_o0, sem_o1):
    cid = lax.axis_index("c")
    sid = lax.axis_index("s")
    wid = sid * 2 + cid
    sem_g = (sem_g0, sem_g1, sem_g2, sem_g3)
    sem_o = (sem_o0, sem_o1)

    pltpu.sync_copy(pos_hbm, pos_v)
    pltpu.sync_copy(idx_hbm.at[:, pl.ds(wid, 1)], idx_v)

    def fire_gather(s, r):
        pltpu.async_copy(
            table_hbm.at[idx_v.at[s // 8, 0, s % 8]],
            rows_v.at[r],
            sem_g[r],
        )

    def drain_gather(r):
        pltpu.make_async_copy(
            table_hbm.at[pl.ds(0, BBLK)], rows_v.at[r], sem_g[r]
        ).wait()

    def drain_store(p):
        pltpu.make_async_copy(
            brick_v.at[p].at[:, :, pl.ds(0, 128)],
            out_hbm.at[0, :, 0],
            sem_o[p],
        ).wait()

    lane_d = [jnp.arange(k * LANES, (k + 1) * LANES, dtype=jnp.int32)
              for k in range(DIM // LANES)]
    dt_vecs = [d // 8 for d in lane_d]
    di_vecs = [d % 8 for d in lane_d]

    def build_brick(s, r, p):
        rows = rows_v.at[r]
        brick = brick_v.at[p]
        pvs = [pos_v[s, pl.ds(k * LANES, LANES)] for k in range(DIM // LANES)]

        @plsc.parallel_loop(0, BBLK, unroll=16)
        def _(b):
            col = jnp.full((LANES,), b, dtype=jnp.int32)
            for k in range(DIM // LANES):
                v = rows[b, pl.ds(k * LANES, LANES)] + pvs[k]
                plsc.store_scatter(brick, [dt_vecs[k], di_vecs[k], col], v)

    for s0 in range(3):
        fire_gather(s0, s0)

    @pl.loop(0, SEQ, step=NGBUF)
    def _(ss):
        for r in range(NGBUF):
            s = ss + r
            p = r % 2

            @pl.when(s + 3 < SEQ)
            def _prefetch():
                fire_gather(s + 3, (r + 3) % NGBUF)

            drain_gather(r)

            @pl.when(s >= 2)
            def _reclaim():
                drain_store(p)

            build_brick(s, r, p)
            pltpu.async_copy(
                brick_v.at[p].at[:, :, pl.ds(0, 128)],
                out_hbm.at[s, :, wid],
                sem_o[p],
            )

    drain_store(0)
    drain_store(1)


_sc_embed = functools.partial(
    pl.kernel,
    out_type=jax.ShapeDtypeStruct((SEQ, DIM // 8, NW, 8, 128), jnp.float32),
    mesh=plsc.VectorSubcoreMesh(
        core_axis_name="c", subcore_axis_name="s", num_cores=2, num_subcores=16
    ),
    scratch_types=[
        pltpu.VMEM((SEQ // 8, 1, 8, 128), jnp.int32),
        pltpu.VMEM((NGBUF, BBLK, DIM), jnp.float32),
        pltpu.VMEM((2, DIM // 8, 8, PITCH), jnp.float32),
        pltpu.VMEM((SEQ, DIM), jnp.float32),
        pltpu.SemaphoreType.DMA,
        pltpu.SemaphoreType.DMA,
        pltpu.SemaphoreType.DMA,
        pltpu.SemaphoreType.DMA,
        pltpu.SemaphoreType.DMA,
        pltpu.SemaphoreType.DMA,
    ],
    compiler_params=pltpu.CompilerParams(
        use_tc_tiling_on_sc=False, needs_layout_passes=False
    ),
)(_sc_body)


def kernel(inputs, table, pos_table):
    idx = (inputs.transpose(1, 0).reshape(SEQ // 8, 8, NW, 128)
           .transpose(0, 2, 1, 3))
    q = _sc_embed(idx, table, pos_table)
    return q.transpose(2, 4, 0, 1, 3).reshape(BATCH, SEQ, DIM)
